# baseline (speedup 1.0000x reference)
import functools

import jax
import jax.numpy as jnp
from jax.experimental import pallas as pl


def _readout_block(e_ref, wl_ref, wn1_ref, bn1_ref, wn2_ref, bias_ref,
                   ones_ref, out_ref):
    bn = e_ref.shape[0] // 4
    x0 = e_ref[pl.Slice(0, bn, 4), :]
    x1 = e_ref[pl.Slice(1, bn, 4), :]
    x2 = e_ref[pl.Slice(2, bn, 4), :]
    x3 = e_ref[pl.Slice(3, bn, 4), :]
    wlb = wl_ref[...].astype(jnp.bfloat16)
    lin = (jnp.dot(x0.astype(jnp.bfloat16), wlb[0:1, :].T,
                   preferred_element_type=jnp.float32)
           + jnp.dot(x1.astype(jnp.bfloat16), wlb[1:2, :].T,
                     preferred_element_type=jnp.float32)
           + jnp.dot(x2.astype(jnp.bfloat16), wlb[2:3, :].T,
                     preferred_element_type=jnp.float32))
    h = jnp.dot(x3.astype(jnp.bfloat16),
                wn1_ref[...].astype(jnp.bfloat16),
                preferred_element_type=jnp.float32) + bn1_ref[...]
    h = h * jax.nn.sigmoid(h)
    mlp = jnp.dot(h.astype(jnp.bfloat16),
                  wn2_ref[...].astype(jnp.bfloat16),
                  preferred_element_type=jnp.float32)
    out_ref[...] = (lin + mlp)[:, 0] + bias_ref[0, 0]


@functools.partial(jax.jit, static_argnames=())
def kernel(embedding_0, Wlin, blin, Wn1, bn1, Wn2, bn2):
    N, IN, _, R = embedding_0.shape
    H = Wn1.shape[1]
    e = embedding_0.transpose(0, 3, 2, 1).reshape(N * R, IN)

    wl = Wlin[:, :, 0]
    bias = (jnp.sum(blin) + bn2[0]).reshape(1, 1)
    ones = jnp.ones((IN, 1), jnp.float32)

    Bn = 8192
    grid = (pl.cdiv(N, Bn),)
    out = pl.pallas_call(
        _readout_block,
        grid=grid,
        in_specs=[
            pl.BlockSpec((R * Bn, IN), lambda i: (i, 0)),
            pl.BlockSpec((R - 1, IN), lambda i: (0, 0)),
            pl.BlockSpec((IN, H), lambda i: (0, 0)),
            pl.BlockSpec((1, H), lambda i: (0, 0)),
            pl.BlockSpec((H, 1), lambda i: (0, 0)),
            pl.BlockSpec((1, 1), lambda i: (0, 0)),
            pl.BlockSpec((IN, 1), lambda i: (0, 0)),
        ],
        out_specs=pl.BlockSpec((Bn,), lambda i: (i,)),
        out_shape=jax.ShapeDtypeStruct((N,), jnp.float32),
    )(e, wl, Wn1, bn1.reshape(1, H), Wn2, bias, ones)
    return out


# speedup vs baseline: 1.1548x; 1.1548x over previous
import functools

import jax
import jax.numpy as jnp
from jax.experimental import pallas as pl


def _readout_block(e_ref, wl_ref, wn1_ref, bn1_ref, wn2_ref, bias_ref,
                   ones_ref, out_ref):
    bn = e_ref.shape[0] // 4
    x0 = e_ref[pl.Slice(0, bn, 4), :]
    x1 = e_ref[pl.Slice(1, bn, 4), :]
    x2 = e_ref[pl.Slice(2, bn, 4), :]
    x3 = e_ref[pl.Slice(3, bn, 4), :]
    lincomb = (x0 * wl_ref[0:1, :] + x1 * wl_ref[1:2, :]
               + x2 * wl_ref[2:3, :])
    lin = jnp.dot(lincomb, ones_ref[...],
                  preferred_element_type=jnp.float32)
    h = jnp.dot(x3, wn1_ref[...],
                preferred_element_type=jnp.float32) + bn1_ref[...]
    h = h * jax.nn.sigmoid(h)
    mlp = jnp.dot(h, wn2_ref[...], preferred_element_type=jnp.float32)
    out_ref[...] = (lin + mlp)[:, 0] + bias_ref[0, 0]


@functools.partial(jax.jit, static_argnames=())
def kernel(embedding_0, Wlin, blin, Wn1, bn1, Wn2, bn2):
    N, IN, _, R = embedding_0.shape
    H = Wn1.shape[1]
    e = embedding_0.transpose(0, 3, 2, 1).reshape(N * R, IN)

    wl = Wlin[:, :, 0]
    bias = (jnp.sum(blin) + bn2[0]).reshape(1, 1)
    ones = jnp.ones((IN, 1), jnp.float32)

    Bn = 9216
    grid = (pl.cdiv(N, Bn),)
    out = pl.pallas_call(
        _readout_block,
        grid=grid,
        in_specs=[
            pl.BlockSpec((R * Bn, IN), lambda i: (i, 0)),
            pl.BlockSpec((R - 1, IN), lambda i: (0, 0)),
            pl.BlockSpec((IN, H), lambda i: (0, 0)),
            pl.BlockSpec((1, H), lambda i: (0, 0)),
            pl.BlockSpec((H, 1), lambda i: (0, 0)),
            pl.BlockSpec((1, 1), lambda i: (0, 0)),
            pl.BlockSpec((IN, 1), lambda i: (0, 0)),
        ],
        out_specs=pl.BlockSpec((Bn,), lambda i: (i,)),
        out_shape=jax.ShapeDtypeStruct((N,), jnp.float32),
    )(e, wl, Wn1, bn1.reshape(1, H), Wn2, bias, ones)
    return out
